# baseline (speedup 1.0000x reference)
import dataclasses
import functools

import jax
import jax.numpy as jnp
from jax import lax
from jax.experimental import pallas as pl
from jax.experimental.pallas import tpu as pltpu
from jax.experimental.pallas import tpu_sc as plsc

B = 128
NG = 2
PPG = 20000
D = 3
N = NG * PPG
R = 313
NPAD = R * 128
STD = 0.1
NW = 32
BPW = B // NW
CH = 1600

_f32 = jnp.float32
_i32 = jnp.int32

_cp_sc = pltpu.CompilerParams()
if "needs_layout_passes" in pltpu.CompilerParams.__dataclass_fields__:
    _cp_sc = dataclasses.replace(_cp_sc, needs_layout_passes=False)


def _sum_body(w_ref, rec_ref):
    def chunk(c, S):
        def step(r, acc):
            base = (c * 500 + r) * 8
            acc = acc + w_ref[pl.ds(base, 8), :]
            return acc + w_ref[pl.ds(20000 + base, 8), :]

        acc = lax.fori_loop(0, 500, step, jnp.zeros((8, B), _f32))
        t1 = acc[0:4] + acc[4:8]
        t2 = t1[0:2] + t1[2:4]
        return S + (t2[0:1] + t2[1:2])

    S = lax.fori_loop(0, 5, chunk, jnp.zeros((1, B), _f32))
    rec_ref[...] = _f32(1.0) / S


def _weight_recip(wT):
    return pl.pallas_call(
        _sum_body,
        out_shape=jax.ShapeDtypeStruct((1, B), _f32),
    )(wT)


def _cdf_body(x_ref, rec_ref, o_ref, s384_ref):
    o_ref[...] = x_ref[...] * rec_ref[...].reshape(1, 1, B)

    def scan_step(t, _):
        o_ref[pl.ds(t, 1)] = o_ref[pl.ds(t, 1)] + o_ref[pl.ds(t - 1, 1)]
        return 0

    lax.fori_loop(1, 128, scan_step, 0)

    def q_chain(base, n_r, off2):
        chain0 = o_ref[127, pl.ds(base, 1), :]
        s384_ref[pl.ds(base, 1), :] = chain0 + off2

        def step(r, chain):
            c2 = chain + o_ref[127, pl.ds(r, 1), :]
            s384_ref[pl.ds(r, 1), :] = c2 + off2
            return c2

        return lax.fori_loop(base + 1, base + n_r, step, chain0)

    t20 = q_chain(0, 128, jnp.zeros((1, B), _f32))
    t21 = q_chain(128, 128, t20)
    q_chain(256, R - 256, t20 + t21)

    def add_off(t, _):
        o_ref[pl.ds(t, 1), 1:R, :] = (
            o_ref[pl.ds(t, 1), 1:R, :] + s384_ref[0:R - 1, :]
        )
        return 0

    lax.fori_loop(0, 128, add_off, 0)


def _cdf(Xw, rec):
    return pl.pallas_call(
        _cdf_body,
        out_shape=jax.ShapeDtypeStruct((128, R, B), _f32),
        scratch_shapes=[pltpu.VMEM((R, B), _f32)],
    )(Xw, rec)


def _pos_body(u_ref, o_ref):
    c = _f32(1.0) / _f32(N)
    ii = lax.broadcasted_iota(_i32, (8, N), 1).astype(_f32)
    o_ref[:, :N] = (u_ref[...] + ii) * c
    o_ref[:, N:] = jnp.full((8, NPAD - N), 2.0, _f32)


def _pos(u2):
    return pl.pallas_call(
        _pos_body,
        grid=(B // 8,),
        in_specs=[pl.BlockSpec((8, N), lambda i: (i, 0))],
        out_specs=pl.BlockSpec((8, NPAD), lambda i: (i, 0)),
        out_shape=jax.ShapeDtypeStruct((B, NPAD), _f32),
    )(u2)


def _hist(cdf, posP):
    mesh = plsc.VectorSubcoreMesh(core_axis_name="c", subcore_axis_name="s")

    @functools.partial(
        pl.kernel,
        out_type=jax.ShapeDtypeStruct((B, NPAD), _i32),
        mesh=mesh,
        compiler_params=_cp_sc,
        scratch_types=[
            pltpu.VMEM((NPAD,), _f32),
            pltpu.VMEM((NPAD,), _f32),
            pltpu.VMEM((NPAD,), _i32),
        ],
    )
    def hist_kernel(cdf_hbm, pos_hbm, out_hbm, pbuf, cbuf, hbuf):
        wid = lax.axis_index("s") * 2 + lax.axis_index("c")

        @pl.loop(0, BPW)
        def _batch(bi):
            b = bi * NW + wid
            pltpu.sync_copy(pos_hbm.at[b], pbuf)
            pltpu.sync_copy(cdf_hbm.at[b], cbuf)

            @pl.loop(0, NPAD, step=16)
            def _zero(i):
                hbuf[pl.ds(i, 16)] = jnp.zeros((16,), _i32)

            one = jnp.ones((16,), _i32)
            zero = jnp.zeros((16,), _i32)

            @pl.loop(0, N, step=16)
            def _scan(j):
                c16 = cbuf[pl.ds(j, 16)]
                k = (c16 * _f32(N)).astype(_i32)
                k = jnp.minimum(k, N - 1)
                km1 = jnp.maximum(k - 1, 0)
                g0 = plsc.load_gather(pbuf, [km1])
                g1 = plsc.load_gather(pbuf, [k])
                g2 = plsc.load_gather(pbuf, [k + 1])
                i0 = jnp.where(k == 0, one, jnp.where(g0 <= c16, one, zero))
                i1 = jnp.where(g1 <= c16, one, zero)
                i2 = jnp.where(g2 <= c16, one, zero)
                m = k - 1 + i0 + i1 + i2
                plsc.addupdate_scatter(hbuf, [m], one)

            pltpu.sync_copy(hbuf, out_hbm.at[b])

    return hist_kernel(cdf, posP)


def _idx_body(h_ref, o_ref):
    x = h_ref[...]
    for dd in (1, 2, 4, 8, 16, 32, 64):
        x = x + jnp.concatenate(
            [jnp.zeros((8, R, dd), _i32), x[:, :, :-dd]], axis=2)
    t = x[:, :, 127]
    for dd in (1, 2, 4, 8, 16, 32, 64, 128, 256):
        t = t + jnp.concatenate(
            [jnp.zeros((8, dd), _i32), t[:, :-dd]], axis=1)
    offe = jnp.concatenate([jnp.zeros((8, 1), _i32), t[:, :-1]], axis=1)
    o_ref[...] = jnp.minimum(x + offe[:, :, None], N - 1)


def _indices(hist3):
    return pl.pallas_call(
        _idx_body,
        grid=(B // 8,),
        in_specs=[pl.BlockSpec((8, R, 128), lambda i: (i, 0, 0))],
        out_specs=pl.BlockSpec((8, R, 128), lambda i: (i, 0, 0)),
        out_shape=jax.ShapeDtypeStruct((B, R, 128), _i32),
    )(hist3)


NCH = N // CH
TLEN = 120096
ILEN = 1664
OROW = 4864


def _gather(idx_flat, part_flat):
    mesh = plsc.VectorSubcoreMesh(core_axis_name="c", subcore_axis_name="s")

    @functools.partial(
        pl.kernel,
        out_type=jax.ShapeDtypeStruct((B * NCH * OROW,), _f32),
        mesh=mesh,
        compiler_params=_cp_sc,
        scratch_types=[
            pltpu.VMEM((TLEN,), _f32),
            pltpu.VMEM((ILEN,), _i32),
            pltpu.VMEM((OROW,), _f32),
        ],
    )
    def gather_kernel(idx_hbm, part_hbm, out_hbm, tbuf, ibuf, gbuf):
        wid = lax.axis_index("s") * 2 + lax.axis_index("c")
        lane = lax.iota(_i32, 16)

        @pl.loop(0, BPW)
        def _batch(bi):
            b = bi * NW + wid
            poff = b * (N * D)
            prem = jnp.bitwise_and(poff, 127)
            pstart = pl.multiple_of(poff - prem, 128)
            pltpu.sync_copy(part_hbm.at[pl.ds(pstart, TLEN)], tbuf)

            @pl.loop(0, NCH)
            def _chunk(c):
                ioff = b * NPAD + c * CH
                irem = jnp.bitwise_and(ioff, 127)
                istart = pl.multiple_of(ioff - irem, 128)
                pltpu.sync_copy(idx_hbm.at[pl.ds(istart, ILEN)], ibuf)

                @pl.loop(0, CH, step=16)
                def _vec(i):
                    k16 = ibuf[pl.ds(pl.multiple_of(i + irem, 16), 16)]
                    e0 = k16 * 3 + prem
                    s0 = (lane + i) * 3
                    for comp in range(D):
                        g = plsc.load_gather(tbuf, [e0 + comp])
                        plsc.store_scatter(gbuf, [s0 + comp], g)

                pltpu.sync_copy(
                    gbuf, out_hbm.at[pl.ds((b * NCH + c) * OROW, OROW)])

    return gather_kernel(idx_flat, part_flat)


def _noise_body(g_ref, n_ref, o_ref):
    o_ref[...] = g_ref[:, :CH * D] + _f32(STD) * n_ref[...]


def _add_noise(gpad, n2):
    return pl.pallas_call(
        _noise_body,
        grid=(B * NCH // 8,),
        in_specs=[pl.BlockSpec((8, OROW), lambda i: (i, 0)),
                  pl.BlockSpec((8, CH * D), lambda i: (i, 0))],
        out_specs=pl.BlockSpec((8, CH * D), lambda i: (i, 0)),
        out_shape=jax.ShapeDtypeStruct((B * NCH, CH * D), _f32),
    )(gpad, n2)


def kernel(particles, weights, u, noise):
    w = weights.reshape(B, N)
    wT = w.T
    rec = _weight_recip(wT)

    wpad = jnp.pad(w, ((0, 0), (0, NPAD - N)))
    Xw = wpad.reshape(B, R, 128).transpose(2, 1, 0)
    cdfX = _cdf(Xw, rec)
    cdf = cdfX.transpose(2, 1, 0).reshape(B, NPAD)

    posP = _pos(u.reshape(B, N))
    hist = _hist(cdf, posP)
    idx = _indices(hist.reshape(B, R, 128))
    idx_flat = idx.reshape(-1)

    part_flat = particles.reshape(-1)
    gflat = _gather(idx_flat, part_flat)
    out = _add_noise(gflat.reshape(B * NCH, OROW),
                     noise.reshape(B * NCH, CH * D))
    return out.reshape(B, NG, PPG, D)


# speedup vs baseline: 1.6243x; 1.6243x over previous
import dataclasses
import functools

import jax
import jax.numpy as jnp
from jax import lax
from jax.experimental import pallas as pl
from jax.experimental.pallas import tpu as pltpu
from jax.experimental.pallas import tpu_sc as plsc

B = 128
NG = 2
PPG = 20000
D = 3
N = NG * PPG
R = 313
NPAD = R * 128
STD = 0.1
NW = 32
BPW = B // NW
CH = 1600

_f32 = jnp.float32
_i32 = jnp.int32

_cp_sc = pltpu.CompilerParams()
if "needs_layout_passes" in pltpu.CompilerParams.__dataclass_fields__:
    _cp_sc = dataclasses.replace(_cp_sc, needs_layout_passes=False)


def _sum_body(w_ref, rec_ref):
    def chunk(c, S):
        def step(r, acc):
            base = (c * 500 + r) * 8
            acc = acc + w_ref[pl.ds(base, 8), :]
            return acc + w_ref[pl.ds(20000 + base, 8), :]

        acc = lax.fori_loop(0, 500, step, jnp.zeros((8, B), _f32))
        t1 = acc[0:4] + acc[4:8]
        t2 = t1[0:2] + t1[2:4]
        return S + (t2[0:1] + t2[1:2])

    S = lax.fori_loop(0, 5, chunk, jnp.zeros((1, B), _f32))
    rec_ref[...] = _f32(1.0) / S


def _weight_recip(wT):
    return pl.pallas_call(
        _sum_body,
        out_shape=jax.ShapeDtypeStruct((1, B), _f32),
    )(wT)


def _cdf_body(x_ref, rec_ref, o_ref, s384_ref):
    o_ref[...] = x_ref[...] * rec_ref[...].reshape(1, 1, B)

    def scan_step(t, _):
        o_ref[pl.ds(t, 1)] = o_ref[pl.ds(t, 1)] + o_ref[pl.ds(t - 1, 1)]
        return 0

    lax.fori_loop(1, 128, scan_step, 0)

    def q_chain(base, n_r, off2):
        chain0 = o_ref[127, pl.ds(base, 1), :]
        s384_ref[pl.ds(base, 1), :] = chain0 + off2

        def step(r, chain):
            c2 = chain + o_ref[127, pl.ds(r, 1), :]
            s384_ref[pl.ds(r, 1), :] = c2 + off2
            return c2

        return lax.fori_loop(base + 1, base + n_r, step, chain0)

    t20 = q_chain(0, 128, jnp.zeros((1, B), _f32))
    t21 = q_chain(128, 128, t20)
    q_chain(256, R - 256, t20 + t21)

    def add_off(t, _):
        o_ref[pl.ds(t, 1), 1:R, :] = (
            o_ref[pl.ds(t, 1), 1:R, :] + s384_ref[0:R - 1, :]
        )
        return 0

    lax.fori_loop(0, 128, add_off, 0)


def _cdf(Xw, rec):
    return pl.pallas_call(
        _cdf_body,
        out_shape=jax.ShapeDtypeStruct((128, R, B), _f32),
        scratch_shapes=[pltpu.VMEM((R, B), _f32)],
    )(Xw, rec)


def _pos_body(u_ref, o_ref):
    c = _f32(1.0) / _f32(N)
    ii = lax.broadcasted_iota(_i32, (8, N), 1).astype(_f32)
    o_ref[:, :N] = (u_ref[...] + ii) * c
    o_ref[:, N:] = jnp.full((8, NPAD - N), 2.0, _f32)


def _pos(u2):
    return pl.pallas_call(
        _pos_body,
        grid=(B // 8,),
        in_specs=[pl.BlockSpec((8, N), lambda i: (i, 0))],
        out_specs=pl.BlockSpec((8, NPAD), lambda i: (i, 0)),
        out_shape=jax.ShapeDtypeStruct((B, NPAD), _f32),
    )(u2)


def _hist(cdf, posP):
    mesh = plsc.VectorSubcoreMesh(core_axis_name="c", subcore_axis_name="s")

    @functools.partial(
        pl.kernel,
        out_type=jax.ShapeDtypeStruct((B, NPAD), _i32),
        mesh=mesh,
        compiler_params=_cp_sc,
        scratch_types=[
            pltpu.VMEM((NPAD,), _f32),
            pltpu.VMEM((NPAD,), _f32),
            pltpu.VMEM((NPAD,), _i32),
        ],
    )
    def hist_kernel(cdf_hbm, pos_hbm, out_hbm, pbuf, cbuf, hbuf):
        wid = lax.axis_index("s") * 2 + lax.axis_index("c")

        @pl.loop(0, BPW)
        def _batch(bi):
            b = bi * NW + wid
            pltpu.sync_copy(pos_hbm.at[b], pbuf)
            pltpu.sync_copy(cdf_hbm.at[b], cbuf)

            @pl.loop(0, NPAD, step=16)
            def _zero(i):
                hbuf[pl.ds(i, 16)] = jnp.zeros((16,), _i32)

            one = jnp.ones((16,), _i32)
            zero = jnp.zeros((16,), _i32)

            @pl.loop(0, N, step=16)
            def _scan(j):
                c16 = cbuf[pl.ds(j, 16)]
                k = (c16 * _f32(N)).astype(_i32)
                k = jnp.minimum(k, N - 1)
                km1 = jnp.maximum(k - 1, 0)
                g0 = plsc.load_gather(pbuf, [km1])
                g1 = plsc.load_gather(pbuf, [k])
                g2 = plsc.load_gather(pbuf, [k + 1])
                i0 = jnp.where(k == 0, one, jnp.where(g0 <= c16, one, zero))
                i1 = jnp.where(g1 <= c16, one, zero)
                i2 = jnp.where(g2 <= c16, one, zero)
                m = k - 1 + i0 + i1 + i2
                plsc.addupdate_scatter(hbuf, [m], one)

            pltpu.sync_copy(hbuf, out_hbm.at[b])

    return hist_kernel(cdf, posP)


def _idx_body(h_ref, o_ref):
    x = h_ref[...]
    for dd in (1, 2, 4, 8, 16, 32, 64):
        x = x + jnp.concatenate(
            [jnp.zeros((8, R, dd), _i32), x[:, :, :-dd]], axis=2)
    t = x[:, :, 127]
    for dd in (1, 2, 4, 8, 16, 32, 64, 128, 256):
        t = t + jnp.concatenate(
            [jnp.zeros((8, dd), _i32), t[:, :-dd]], axis=1)
    offe = jnp.concatenate([jnp.zeros((8, 1), _i32), t[:, :-1]], axis=1)
    o_ref[...] = jnp.minimum(x + offe[:, :, None], N - 1)


def _indices(hist3):
    return pl.pallas_call(
        _idx_body,
        grid=(B // 8,),
        in_specs=[pl.BlockSpec((8, R, 128), lambda i: (i, 0, 0))],
        out_specs=pl.BlockSpec((8, R, 128), lambda i: (i, 0, 0)),
        out_shape=jax.ShapeDtypeStruct((B, R, 128), _i32),
    )(hist3)


NCH = N // CH
TROW = 120064
IROW = 1664
OROW = 4864


def _gather(idx_flat, part_flat):
    mesh = plsc.VectorSubcoreMesh(core_axis_name="c", subcore_axis_name="s")

    @functools.partial(
        pl.kernel,
        out_type=jax.ShapeDtypeStruct((B * NCH, OROW), _f32),
        mesh=mesh,
        compiler_params=_cp_sc,
        scratch_types=[
            pltpu.VMEM((TROW,), _f32),
            pltpu.VMEM((IROW,), _i32),
            pltpu.VMEM((OROW,), _f32),
        ],
    )
    def gather_kernel(idx_hbm, part_hbm, out_hbm, tbuf, ibuf, gbuf):
        wid = lax.axis_index("s") * 2 + lax.axis_index("c")
        lane = lax.iota(_i32, 16)

        @pl.loop(0, BPW)
        def _batch(bi):
            b = bi * NW + wid
            pltpu.sync_copy(part_hbm.at[b], tbuf)

            @pl.loop(0, NCH)
            def _chunk(c):
                pltpu.sync_copy(idx_hbm.at[b * NCH + c], ibuf)

                @pl.loop(0, CH, step=16)
                def _vec(i):
                    k16 = ibuf[pl.ds(i, 16)]
                    e0 = k16 * 3
                    s0 = (lane + i) * 3
                    for comp in range(D):
                        g = plsc.load_gather(tbuf, [e0 + comp])
                        plsc.store_scatter(gbuf, [s0 + comp], g)

                pltpu.sync_copy(gbuf, out_hbm.at[b * NCH + c])

    return gather_kernel(idx_flat, part_flat)


def _noise_body(g_ref, n_ref, o_ref):
    o_ref[...] = g_ref[:, :CH * D] + _f32(STD) * n_ref[...]


def _add_noise(gpad, n2):
    return pl.pallas_call(
        _noise_body,
        grid=(B * NCH // 8,),
        in_specs=[pl.BlockSpec((8, OROW), lambda i: (i, 0)),
                  pl.BlockSpec((8, CH * D), lambda i: (i, 0))],
        out_specs=pl.BlockSpec((8, CH * D), lambda i: (i, 0)),
        out_shape=jax.ShapeDtypeStruct((B * NCH, CH * D), _f32),
    )(gpad, n2)


def kernel(particles, weights, u, noise):
    w = weights.reshape(B, N)
    wT = w.T
    rec = _weight_recip(wT)

    wpad = jnp.pad(w, ((0, 0), (0, NPAD - N)))
    Xw = wpad.reshape(B, R, 128).transpose(2, 1, 0)
    cdfX = _cdf(Xw, rec)
    cdf = cdfX.transpose(2, 1, 0).reshape(B, NPAD)

    posP = _pos(u.reshape(B, N))
    hist = _hist(cdf, posP)
    idx = _indices(hist.reshape(B, R, 128))
    idx2 = idx.reshape(B, NPAD)[:, :N].reshape(B * NCH, CH)
    idx_pad = jnp.pad(idx2, ((0, 0), (0, IROW - CH)))

    part2 = particles.reshape(B, N * D)
    part_pad = jnp.pad(part2, ((0, 0), (0, TROW - N * D)))

    gpad = _gather(idx_pad, part_pad)
    out = _add_noise(gpad, noise.reshape(B * NCH, CH * D))
    return out.reshape(B, NG, PPG, D)


# speedup vs baseline: 26.8160x; 16.5094x over previous
import dataclasses
import functools

import jax
import jax.numpy as jnp
from jax import lax
from jax.experimental import pallas as pl
from jax.experimental.pallas import tpu as pltpu
from jax.experimental.pallas import tpu_sc as plsc

B = 128
NG = 2
PPG = 20000
D = 3
N = NG * PPG
R = 313
NPAD = R * 128
STD = 0.1
NW = 32
BPW = B // NW
CH = 1600

_f32 = jnp.float32
_i32 = jnp.int32

_cp_sc = pltpu.CompilerParams()
if "needs_layout_passes" in pltpu.CompilerParams.__dataclass_fields__:
    _cp_sc = dataclasses.replace(_cp_sc, needs_layout_passes=False)


def _sum_body(w_ref, rec_ref):
    def chunk(c, S):
        def step(r, acc):
            base = (c * 500 + r) * 8
            acc = acc + w_ref[pl.ds(base, 8), :]
            return acc + w_ref[pl.ds(20000 + base, 8), :]

        acc = lax.fori_loop(0, 500, step, jnp.zeros((8, B), _f32))
        t1 = acc[0:4] + acc[4:8]
        t2 = t1[0:2] + t1[2:4]
        return S + (t2[0:1] + t2[1:2])

    S = lax.fori_loop(0, 5, chunk, jnp.zeros((1, B), _f32))
    rec_ref[...] = _f32(1.0) / S


def _weight_recip(wT):
    return pl.pallas_call(
        _sum_body,
        out_shape=jax.ShapeDtypeStruct((1, B), _f32),
    )(wT)


def _cdf_body(x_ref, rec_ref, o_ref, s384_ref):
    o_ref[...] = x_ref[...] * rec_ref[...].reshape(1, 1, B)

    def scan_step(t, _):
        o_ref[pl.ds(t, 1)] = o_ref[pl.ds(t, 1)] + o_ref[pl.ds(t - 1, 1)]
        return 0

    lax.fori_loop(1, 128, scan_step, 0)

    def q_chain(base, n_r, off2):
        chain0 = o_ref[127, pl.ds(base, 1), :]
        s384_ref[pl.ds(base, 1), :] = chain0 + off2

        def step(r, chain):
            c2 = chain + o_ref[127, pl.ds(r, 1), :]
            s384_ref[pl.ds(r, 1), :] = c2 + off2
            return c2

        return lax.fori_loop(base + 1, base + n_r, step, chain0)

    t20 = q_chain(0, 128, jnp.zeros((1, B), _f32))
    t21 = q_chain(128, 128, t20)
    q_chain(256, R - 256, t20 + t21)

    def add_off(t, _):
        o_ref[pl.ds(t, 1), 1:R, :] = (
            o_ref[pl.ds(t, 1), 1:R, :] + s384_ref[0:R - 1, :]
        )
        return 0

    lax.fori_loop(0, 128, add_off, 0)


def _cdf(Xw, rec):
    return pl.pallas_call(
        _cdf_body,
        out_shape=jax.ShapeDtypeStruct((128, R, B), _f32),
        scratch_shapes=[pltpu.VMEM((R, B), _f32)],
    )(Xw, rec)


def _pos_body(u_ref, o_ref):
    c = _f32(1.0) / _f32(N)
    ii = lax.broadcasted_iota(_i32, (8, N), 1).astype(_f32)
    o_ref[:, :N] = (u_ref[...] + ii) * c
    o_ref[:, N:] = jnp.full((8, NPAD - N), 2.0, _f32)


def _pos(u2):
    return pl.pallas_call(
        _pos_body,
        grid=(B // 8,),
        in_specs=[pl.BlockSpec((8, N), lambda i: (i, 0))],
        out_specs=pl.BlockSpec((8, NPAD), lambda i: (i, 0)),
        out_shape=jax.ShapeDtypeStruct((B, NPAD), _f32),
    )(u2)


def _hist(cdf, posP):
    mesh = plsc.VectorSubcoreMesh(core_axis_name="c", subcore_axis_name="s")

    @functools.partial(
        pl.kernel,
        out_type=jax.ShapeDtypeStruct((B, NPAD), _i32),
        mesh=mesh,
        compiler_params=_cp_sc,
        scratch_types=[
            pltpu.VMEM((NPAD,), _f32),
            pltpu.VMEM((NPAD,), _f32),
            pltpu.VMEM((NPAD,), _i32),
        ],
    )
    def hist_kernel(cdf_hbm, pos_hbm, out_hbm, pbuf, cbuf, hbuf):
        wid = lax.axis_index("s") * 2 + lax.axis_index("c")

        @pl.loop(0, BPW)
        def _batch(bi):
            b = bi * NW + wid
            pltpu.sync_copy(pos_hbm.at[b], pbuf)
            pltpu.sync_copy(cdf_hbm.at[b], cbuf)

            @pl.loop(0, NPAD, step=16)
            def _zero(i):
                hbuf[pl.ds(i, 16)] = jnp.zeros((16,), _i32)

            one = jnp.ones((16,), _i32)
            zero = jnp.zeros((16,), _i32)

            @pl.loop(0, N, step=16)
            def _scan(j):
                c16 = cbuf[pl.ds(j, 16)]
                k = (c16 * _f32(N)).astype(_i32)
                k = jnp.minimum(k, N - 1)
                km1 = jnp.maximum(k - 1, 0)
                g0 = plsc.load_gather(pbuf, [km1])
                g1 = plsc.load_gather(pbuf, [k])
                g2 = plsc.load_gather(pbuf, [k + 1])
                i0 = jnp.where(k == 0, one, jnp.where(g0 <= c16, one, zero))
                i1 = jnp.where(g1 <= c16, one, zero)
                i2 = jnp.where(g2 <= c16, one, zero)
                m = k - 1 + i0 + i1 + i2
                plsc.addupdate_scatter(hbuf, [m], one)

            pltpu.sync_copy(hbuf, out_hbm.at[b])

    return hist_kernel(cdf, posP)


def _idx_body(h_ref, o_ref):
    x = h_ref[...]
    for dd in (1, 2, 4, 8, 16, 32, 64):
        x = x + jnp.concatenate(
            [jnp.zeros((8, R, dd), _i32), x[:, :, :-dd]], axis=2)
    t = x[:, :, 127]
    for dd in (1, 2, 4, 8, 16, 32, 64, 128, 256):
        t = t + jnp.concatenate(
            [jnp.zeros((8, dd), _i32), t[:, :-dd]], axis=1)
    offe = jnp.concatenate([jnp.zeros((8, 1), _i32), t[:, :-1]], axis=1)
    o_ref[...] = jnp.minimum(x + offe[:, :, None], N - 1)


def _indices(hist3):
    return pl.pallas_call(
        _idx_body,
        grid=(B // 8,),
        in_specs=[pl.BlockSpec((8, R, 128), lambda i: (i, 0, 0))],
        out_specs=pl.BlockSpec((8, R, 128), lambda i: (i, 0, 0)),
        out_shape=jax.ShapeDtypeStruct((B, R, 128), _i32),
    )(hist3)


TROW = 120064
CHG = 2560
NFULL = 15
TAIL = N - NFULL * CHG


def _gather(idx2, part_pad):
    mesh = plsc.VectorSubcoreMesh(core_axis_name="c", subcore_axis_name="s")

    @functools.partial(
        pl.kernel,
        out_type=jax.ShapeDtypeStruct((B, TROW), _f32),
        mesh=mesh,
        compiler_params=_cp_sc,
        scratch_types=[
            pltpu.VMEM((TROW,), _f32),
            pltpu.VMEM((CHG,), _i32),
            pltpu.VMEM((CHG * D,), _f32),
        ],
    )
    def gather_kernel(idx_hbm, part_hbm, out_hbm, tbuf, ibuf, gbuf):
        wid = lax.axis_index("s") * 2 + lax.axis_index("c")
        lane = lax.iota(_i32, 16)

        def do_chunk(b, ioff, nrows, isz, osz):
            pltpu.sync_copy(idx_hbm.at[b].at[pl.ds(ioff, isz)],
                            ibuf.at[pl.ds(0, isz)])

            @pl.loop(0, nrows, step=16)
            def _vec(i):
                k16 = ibuf[pl.ds(i, 16)]
                e0 = k16 * 3
                s0 = (lane + i) * 3
                for comp in range(D):
                    g = plsc.load_gather(tbuf, [e0 + comp])
                    plsc.store_scatter(gbuf, [s0 + comp], g)

            pltpu.sync_copy(gbuf.at[pl.ds(0, osz)],
                            out_hbm.at[b].at[pl.ds(ioff * D, osz)])

        @pl.loop(0, BPW)
        def _batch(bi):
            b = bi * NW + wid
            pltpu.sync_copy(part_hbm.at[b], tbuf)

            @pl.loop(0, NFULL)
            def _chunk(c):
                do_chunk(b, pl.multiple_of(c * CHG, 128), CHG, CHG, CHG * D)

            do_chunk(b, NFULL * CHG, TAIL, 1664, 4864)

    return gather_kernel(idx2, part_pad)


def _noise_body(g_ref, n_ref, o_ref):
    o_ref[...] = g_ref[:, :N * D] + _f32(STD) * n_ref[...]


def _add_noise(gpad, n2):
    return pl.pallas_call(
        _noise_body,
        grid=(B // 8,),
        in_specs=[pl.BlockSpec((8, TROW), lambda i: (i, 0)),
                  pl.BlockSpec((8, N * D), lambda i: (i, 0))],
        out_specs=pl.BlockSpec((8, N * D), lambda i: (i, 0)),
        out_shape=jax.ShapeDtypeStruct((B, N * D), _f32),
    )(gpad, n2)


def kernel(particles, weights, u, noise):
    w = weights.reshape(B, N)
    wT = w.T
    rec = _weight_recip(wT)

    wpad = jnp.pad(w, ((0, 0), (0, NPAD - N)))
    Xw = wpad.reshape(B, R, 128).transpose(2, 1, 0)
    cdfX = _cdf(Xw, rec)
    cdf = cdfX.transpose(2, 1, 0).reshape(B, NPAD)

    posP = _pos(u.reshape(B, N))
    hist = _hist(cdf, posP)
    idx = _indices(hist.reshape(B, R, 128))
    idx2 = idx.reshape(B, NPAD)

    part2 = particles.reshape(B, N * D)
    part_pad = jnp.pad(part2, ((0, 0), (0, TROW - N * D)))

    gpad = _gather(idx2, part_pad)
    out = _add_noise(gpad, noise.reshape(B, N * D))
    return out.reshape(B, NG, PPG, D)


# speedup vs baseline: 27.8975x; 1.0403x over previous
import dataclasses
import functools

import jax
import jax.numpy as jnp
from jax import lax
from jax.experimental import pallas as pl
from jax.experimental.pallas import tpu as pltpu
from jax.experimental.pallas import tpu_sc as plsc

B = 128
NG = 2
PPG = 20000
D = 3
N = NG * PPG
R = 313
NPAD = R * 128
STD = 0.1
NW = 32
BPW = B // NW
CH = 1600

_f32 = jnp.float32
_i32 = jnp.int32

_cp_sc = pltpu.CompilerParams()
if "needs_layout_passes" in pltpu.CompilerParams.__dataclass_fields__:
    _cp_sc = dataclasses.replace(_cp_sc, needs_layout_passes=False)


def _sum_body(w_ref, rec_ref):
    def chunk(c, S):
        def step(r, acc):
            base = (c * 500 + r) * 8
            acc = acc + w_ref[pl.ds(base, 8), :]
            return acc + w_ref[pl.ds(20000 + base, 8), :]

        acc = lax.fori_loop(0, 500, step, jnp.zeros((8, B), _f32))
        t1 = acc[0:4] + acc[4:8]
        t2 = t1[0:2] + t1[2:4]
        return S + (t2[0:1] + t2[1:2])

    S = lax.fori_loop(0, 5, chunk, jnp.zeros((1, B), _f32))
    rec_ref[...] = _f32(1.0) / S


def _weight_recip(wT):
    return pl.pallas_call(
        _sum_body,
        out_shape=jax.ShapeDtypeStruct((1, B), _f32),
    )(wT)


def _cdf_body(x_ref, rec_ref, o_ref, s384_ref):
    o_ref[...] = x_ref[...] * rec_ref[...].reshape(1, 1, B)

    def scan_step(t, _):
        o_ref[pl.ds(t, 1)] = o_ref[pl.ds(t, 1)] + o_ref[pl.ds(t - 1, 1)]
        return 0

    lax.fori_loop(1, 128, scan_step, 0)

    def q_chain(base, n_r, off2):
        chain0 = o_ref[127, pl.ds(base, 1), :]
        s384_ref[pl.ds(base, 1), :] = chain0 + off2

        def step(r, chain):
            c2 = chain + o_ref[127, pl.ds(r, 1), :]
            s384_ref[pl.ds(r, 1), :] = c2 + off2
            return c2

        return lax.fori_loop(base + 1, base + n_r, step, chain0)

    t20 = q_chain(0, 128, jnp.zeros((1, B), _f32))
    t21 = q_chain(128, 128, t20)
    q_chain(256, R - 256, t20 + t21)

    def add_off(t, _):
        o_ref[pl.ds(t, 1), 1:R, :] = (
            o_ref[pl.ds(t, 1), 1:R, :] + s384_ref[0:R - 1, :]
        )
        return 0

    lax.fori_loop(0, 128, add_off, 0)


def _cdf(Xw, rec):
    return pl.pallas_call(
        _cdf_body,
        out_shape=jax.ShapeDtypeStruct((128, R, B), _f32),
        scratch_shapes=[pltpu.VMEM((R, B), _f32)],
    )(Xw, rec)


def _pos_body(u_ref, o_ref):
    c = _f32(1.0) / _f32(N)
    ii = lax.broadcasted_iota(_i32, (8, N), 1).astype(_f32)
    o_ref[:, :N] = (u_ref[...] + ii) * c
    o_ref[:, N:] = jnp.full((8, NPAD - N), 2.0, _f32)


def _pos(u2):
    return pl.pallas_call(
        _pos_body,
        grid=(B // 8,),
        in_specs=[pl.BlockSpec((8, N), lambda i: (i, 0))],
        out_specs=pl.BlockSpec((8, NPAD), lambda i: (i, 0)),
        out_shape=jax.ShapeDtypeStruct((B, NPAD), _f32),
    )(u2)


def _hist(cdf, posP):
    mesh = plsc.VectorSubcoreMesh(core_axis_name="c", subcore_axis_name="s")

    @functools.partial(
        pl.kernel,
        out_type=jax.ShapeDtypeStruct((B, NPAD), _i32),
        mesh=mesh,
        compiler_params=_cp_sc,
        scratch_types=[
            pltpu.VMEM((NPAD,), _f32),
            pltpu.VMEM((NPAD,), _f32),
            pltpu.VMEM((NPAD,), _i32),
        ],
    )
    def hist_kernel(cdf_hbm, pos_hbm, out_hbm, pbuf, cbuf, hbuf):
        wid = lax.axis_index("s") * 2 + lax.axis_index("c")

        @pl.loop(0, BPW)
        def _batch(bi):
            b = bi * NW + wid
            pltpu.sync_copy(pos_hbm.at[b], pbuf)
            pltpu.sync_copy(cdf_hbm.at[b], cbuf)

            @pl.loop(0, NPAD, step=64)
            def _zero(i):
                for t in range(4):
                    hbuf[pl.ds(i + t * 16, 16)] = jnp.zeros((16,), _i32)

            one = jnp.ones((16,), _i32)
            zero = jnp.zeros((16,), _i32)

            @pl.loop(0, N, step=64)
            def _scan(j):
                for t in range(4):
                    c16 = cbuf[pl.ds(j + t * 16, 16)]
                    k = (c16 * _f32(N)).astype(_i32)
                    k = jnp.minimum(k, N - 1)
                    km1 = jnp.maximum(k - 1, 0)
                    g0 = plsc.load_gather(pbuf, [km1])
                    g1 = plsc.load_gather(pbuf, [k])
                    g2 = plsc.load_gather(pbuf, [k + 1])
                    i0 = jnp.where(k == 0, one,
                                   jnp.where(g0 <= c16, one, zero))
                    i1 = jnp.where(g1 <= c16, one, zero)
                    i2 = jnp.where(g2 <= c16, one, zero)
                    m = k - 1 + i0 + i1 + i2
                    plsc.addupdate_scatter(hbuf, [m], one)

            pltpu.sync_copy(hbuf, out_hbm.at[b])

    return hist_kernel(cdf, posP)


def _idx_body(h_ref, o_ref):
    x = h_ref[...]
    for dd in (1, 2, 4, 8, 16, 32, 64):
        x = x + jnp.concatenate(
            [jnp.zeros((8, R, dd), _i32), x[:, :, :-dd]], axis=2)
    t = x[:, :, 127]
    for dd in (1, 2, 4, 8, 16, 32, 64, 128, 256):
        t = t + jnp.concatenate(
            [jnp.zeros((8, dd), _i32), t[:, :-dd]], axis=1)
    offe = jnp.concatenate([jnp.zeros((8, 1), _i32), t[:, :-1]], axis=1)
    o_ref[...] = jnp.minimum(x + offe[:, :, None], N - 1)


def _indices(hist3):
    return pl.pallas_call(
        _idx_body,
        grid=(B // 8,),
        in_specs=[pl.BlockSpec((8, R, 128), lambda i: (i, 0, 0))],
        out_specs=pl.BlockSpec((8, R, 128), lambda i: (i, 0, 0)),
        out_shape=jax.ShapeDtypeStruct((B, R, 128), _i32),
    )(hist3)


TROW = 120064
CHG = 2560
NFULL = 15
TAIL = N - NFULL * CHG


def _gather(idx2, part_pad):
    mesh = plsc.VectorSubcoreMesh(core_axis_name="c", subcore_axis_name="s")

    @functools.partial(
        pl.kernel,
        out_type=jax.ShapeDtypeStruct((B, TROW), _f32),
        mesh=mesh,
        compiler_params=_cp_sc,
        scratch_types=[
            pltpu.VMEM((TROW,), _f32),
            pltpu.VMEM((CHG,), _i32),
            pltpu.VMEM((CHG * D,), _f32),
        ],
    )
    def gather_kernel(idx_hbm, part_hbm, out_hbm, tbuf, ibuf, gbuf):
        wid = lax.axis_index("s") * 2 + lax.axis_index("c")
        lane = lax.iota(_i32, 16)

        def do_chunk(b, ioff, nrows, isz, osz):
            pltpu.sync_copy(idx_hbm.at[b].at[pl.ds(ioff, isz)],
                            ibuf.at[pl.ds(0, isz)])

            @pl.loop(0, nrows, step=64)
            def _vec(i):
                for t in range(4):
                    k16 = ibuf[pl.ds(i + t * 16, 16)]
                    e0 = k16 * 3
                    s0 = (lane + i + t * 16) * 3
                    for comp in range(D):
                        g = plsc.load_gather(tbuf, [e0 + comp])
                        plsc.store_scatter(gbuf, [s0 + comp], g)

            pltpu.sync_copy(gbuf.at[pl.ds(0, osz)],
                            out_hbm.at[b].at[pl.ds(ioff * D, osz)])

        @pl.loop(0, BPW)
        def _batch(bi):
            b = bi * NW + wid
            pltpu.sync_copy(part_hbm.at[b], tbuf)

            @pl.loop(0, NFULL)
            def _chunk(c):
                do_chunk(b, pl.multiple_of(c * CHG, 128), CHG, CHG, CHG * D)

            do_chunk(b, NFULL * CHG, TAIL, 1664, 4864)

    return gather_kernel(idx2, part_pad)


def _noise_body(g_ref, n_ref, o_ref):
    o_ref[...] = g_ref[:, :N * D] + _f32(STD) * n_ref[...]


def _add_noise(gpad, n2):
    return pl.pallas_call(
        _noise_body,
        grid=(B // 8,),
        in_specs=[pl.BlockSpec((8, TROW), lambda i: (i, 0)),
                  pl.BlockSpec((8, N * D), lambda i: (i, 0))],
        out_specs=pl.BlockSpec((8, N * D), lambda i: (i, 0)),
        out_shape=jax.ShapeDtypeStruct((B, N * D), _f32),
    )(gpad, n2)


def kernel(particles, weights, u, noise):
    w = weights.reshape(B, N)
    wT = w.T
    rec = _weight_recip(wT)

    wpad = jnp.pad(w, ((0, 0), (0, NPAD - N)))
    Xw = wpad.reshape(B, R, 128).transpose(2, 1, 0)
    cdfX = _cdf(Xw, rec)
    cdf = cdfX.transpose(2, 1, 0).reshape(B, NPAD)

    posP = _pos(u.reshape(B, N))
    hist = _hist(cdf, posP)
    idx = _indices(hist.reshape(B, R, 128))
    idx2 = idx.reshape(B, NPAD)

    part2 = particles.reshape(B, N * D)
    part_pad = jnp.pad(part2, ((0, 0), (0, TROW - N * D)))

    gpad = _gather(idx2, part_pad)
    out = _add_noise(gpad, noise.reshape(B, N * D))
    return out.reshape(B, NG, PPG, D)


# speedup vs baseline: 34.6117x; 1.2407x over previous
import dataclasses
import functools

import jax
import jax.numpy as jnp
from jax import lax
from jax.experimental import pallas as pl
from jax.experimental.pallas import tpu as pltpu
from jax.experimental.pallas import tpu_sc as plsc

B = 128
NG = 2
PPG = 20000
D = 3
N = NG * PPG
R = 313
NPAD = R * 128
STD = 0.1
NW = 32
BPW = B // NW
CH = 1600

_f32 = jnp.float32
_i32 = jnp.int32

_cp_sc = pltpu.CompilerParams()
if "needs_layout_passes" in pltpu.CompilerParams.__dataclass_fields__:
    _cp_sc = dataclasses.replace(_cp_sc, needs_layout_passes=False)


def _sum_body(w_ref, rec_ref):
    def chunk(c, S):
        def step(r, acc):
            base = (c * 500 + r) * 8
            acc = acc + w_ref[pl.ds(base, 8), :]
            return acc + w_ref[pl.ds(20000 + base, 8), :]

        acc = lax.fori_loop(0, 500, step, jnp.zeros((8, B), _f32))
        t1 = acc[0:4] + acc[4:8]
        t2 = t1[0:2] + t1[2:4]
        return S + (t2[0:1] + t2[1:2])

    S = lax.fori_loop(0, 5, chunk, jnp.zeros((1, B), _f32))
    rec_ref[...] = _f32(1.0) / S


def _weight_recip(wT):
    return pl.pallas_call(
        _sum_body,
        out_shape=jax.ShapeDtypeStruct((1, B), _f32),
    )(wT)


def _cdf_body(x_ref, rec_ref, o_ref, s384_ref):
    o_ref[...] = x_ref[...] * rec_ref[...].reshape(1, 1, B)

    def scan_step(t, _):
        o_ref[pl.ds(t, 1)] = o_ref[pl.ds(t, 1)] + o_ref[pl.ds(t - 1, 1)]
        return 0

    lax.fori_loop(1, 128, scan_step, 0)

    def q_chain(base, n_r, off2):
        chain0 = o_ref[127, pl.ds(base, 1), :]
        s384_ref[pl.ds(base, 1), :] = chain0 + off2

        def step(r, chain):
            c2 = chain + o_ref[127, pl.ds(r, 1), :]
            s384_ref[pl.ds(r, 1), :] = c2 + off2
            return c2

        return lax.fori_loop(base + 1, base + n_r, step, chain0)

    t20 = q_chain(0, 128, jnp.zeros((1, B), _f32))
    t21 = q_chain(128, 128, t20)
    q_chain(256, R - 256, t20 + t21)

    def add_off(t, _):
        o_ref[pl.ds(t, 1), 1:R, :] = (
            o_ref[pl.ds(t, 1), 1:R, :] + s384_ref[0:R - 1, :]
        )
        return 0

    lax.fori_loop(0, 128, add_off, 0)


def _cdf(Xw, rec):
    return pl.pallas_call(
        _cdf_body,
        out_shape=jax.ShapeDtypeStruct((128, R, B), _f32),
        scratch_shapes=[pltpu.VMEM((R, B), _f32)],
    )(Xw, rec)


def _pos_body(u_ref, o_ref):
    c = _f32(1.0) / _f32(N)
    ii = lax.broadcasted_iota(_i32, (8, N), 1).astype(_f32)
    o_ref[:, :N] = (u_ref[...] + ii) * c
    o_ref[:, N:] = jnp.full((8, NPAD - N), 2.0, _f32)


def _pos(u2):
    return pl.pallas_call(
        _pos_body,
        grid=(B // 8,),
        in_specs=[pl.BlockSpec((8, N), lambda i: (i, 0))],
        out_specs=pl.BlockSpec((8, NPAD), lambda i: (i, 0)),
        out_shape=jax.ShapeDtypeStruct((B, NPAD), _f32),
    )(u2)


def _hist(cdf, posP):
    mesh = plsc.VectorSubcoreMesh(core_axis_name="c", subcore_axis_name="s")

    @functools.partial(
        pl.kernel,
        out_type=jax.ShapeDtypeStruct((B, NPAD), _i32),
        mesh=mesh,
        compiler_params=_cp_sc,
        scratch_types=[
            pltpu.VMEM((NPAD,), _f32),
            pltpu.VMEM((NPAD,), _f32),
            pltpu.VMEM((NPAD,), _i32),
        ],
    )
    def hist_kernel(cdf_hbm, pos_hbm, out_hbm, pbuf, cbuf, hbuf):
        wid = lax.axis_index("s") * 2 + lax.axis_index("c")

        @pl.loop(0, BPW)
        def _batch(bi):
            b = bi * NW + wid
            pltpu.sync_copy(pos_hbm.at[b], pbuf)
            pltpu.sync_copy(cdf_hbm.at[b], cbuf)

            @plsc.parallel_loop(0, NPAD, step=16, unroll=4)
            def _zero(i):
                hbuf[pl.ds(i, 16)] = jnp.zeros((16,), _i32)

            one = jnp.ones((16,), _i32)
            zero = jnp.zeros((16,), _i32)

            @plsc.parallel_loop(0, N, step=16, unroll=4)
            def _scan(j):
                c16 = cbuf[pl.ds(j, 16)]
                k = (c16 * _f32(N)).astype(_i32)
                k = jnp.minimum(k, N - 1)
                km1 = jnp.maximum(k - 1, 0)
                g0 = plsc.load_gather(pbuf, [km1])
                g1 = plsc.load_gather(pbuf, [k])
                g2 = plsc.load_gather(pbuf, [k + 1])
                i0 = jnp.where(k == 0, one,
                               jnp.where(g0 <= c16, one, zero))
                i1 = jnp.where(g1 <= c16, one, zero)
                i2 = jnp.where(g2 <= c16, one, zero)
                m = k - 1 + i0 + i1 + i2
                plsc.addupdate_scatter(hbuf, [m], one)

            pltpu.sync_copy(hbuf, out_hbm.at[b])

    return hist_kernel(cdf, posP)


def _idx_body(h_ref, o_ref):
    x = h_ref[...]
    for dd in (1, 2, 4, 8, 16, 32, 64):
        x = x + jnp.concatenate(
            [jnp.zeros((8, R, dd), _i32), x[:, :, :-dd]], axis=2)
    t = x[:, :, 127]
    for dd in (1, 2, 4, 8, 16, 32, 64, 128, 256):
        t = t + jnp.concatenate(
            [jnp.zeros((8, dd), _i32), t[:, :-dd]], axis=1)
    offe = jnp.concatenate([jnp.zeros((8, 1), _i32), t[:, :-1]], axis=1)
    o_ref[...] = jnp.minimum(x + offe[:, :, None], N - 1)


def _indices(hist3):
    return pl.pallas_call(
        _idx_body,
        grid=(B // 8,),
        in_specs=[pl.BlockSpec((8, R, 128), lambda i: (i, 0, 0))],
        out_specs=pl.BlockSpec((8, R, 128), lambda i: (i, 0, 0)),
        out_shape=jax.ShapeDtypeStruct((B, R, 128), _i32),
    )(hist3)


TROW = 120064
CHG = 2560
NFULL = 15
TAIL = N - NFULL * CHG


def _gather(idx2, part_pad):
    mesh = plsc.VectorSubcoreMesh(core_axis_name="c", subcore_axis_name="s")

    @functools.partial(
        pl.kernel,
        out_type=jax.ShapeDtypeStruct((B, TROW), _f32),
        mesh=mesh,
        compiler_params=_cp_sc,
        scratch_types=[
            pltpu.VMEM((TROW,), _f32),
            pltpu.VMEM((CHG,), _i32),
            pltpu.VMEM((CHG * D,), _f32),
        ],
    )
    def gather_kernel(idx_hbm, part_hbm, out_hbm, tbuf, ibuf, gbuf):
        wid = lax.axis_index("s") * 2 + lax.axis_index("c")
        lane = lax.iota(_i32, 16)

        def do_chunk(b, ioff, nrows, isz, osz):
            pltpu.sync_copy(idx_hbm.at[b].at[pl.ds(ioff, isz)],
                            ibuf.at[pl.ds(0, isz)])

            @plsc.parallel_loop(0, nrows, step=16, unroll=4)
            def _vec(i):
                k16 = ibuf[pl.ds(i, 16)]
                e0 = k16 * 3
                s0 = (lane + i) * 3
                for comp in range(D):
                    g = plsc.load_gather(tbuf, [e0 + comp])
                    plsc.store_scatter(gbuf, [s0 + comp], g)

            pltpu.sync_copy(gbuf.at[pl.ds(0, osz)],
                            out_hbm.at[b].at[pl.ds(ioff * D, osz)])

        @pl.loop(0, BPW)
        def _batch(bi):
            b = bi * NW + wid
            pltpu.sync_copy(part_hbm.at[b], tbuf)

            @pl.loop(0, NFULL)
            def _chunk(c):
                do_chunk(b, pl.multiple_of(c * CHG, 128), CHG, CHG, CHG * D)

            do_chunk(b, NFULL * CHG, TAIL, 1664, 4864)

    return gather_kernel(idx2, part_pad)


def _noise_body(g_ref, n_ref, o_ref):
    o_ref[...] = g_ref[:, :N * D] + _f32(STD) * n_ref[...]


def _add_noise(gpad, n2):
    return pl.pallas_call(
        _noise_body,
        grid=(B // 8,),
        in_specs=[pl.BlockSpec((8, TROW), lambda i: (i, 0)),
                  pl.BlockSpec((8, N * D), lambda i: (i, 0))],
        out_specs=pl.BlockSpec((8, N * D), lambda i: (i, 0)),
        out_shape=jax.ShapeDtypeStruct((B, N * D), _f32),
    )(gpad, n2)


def kernel(particles, weights, u, noise):
    w = weights.reshape(B, N)
    wT = w.T
    rec = _weight_recip(wT)

    wpad = jnp.pad(w, ((0, 0), (0, NPAD - N)))
    Xw = wpad.reshape(B, R, 128).transpose(2, 1, 0)
    cdfX = _cdf(Xw, rec)
    cdf = cdfX.transpose(2, 1, 0).reshape(B, NPAD)

    posP = _pos(u.reshape(B, N))
    hist = _hist(cdf, posP)
    idx = _indices(hist.reshape(B, R, 128))
    idx2 = idx.reshape(B, NPAD)

    part2 = particles.reshape(B, N * D)
    part_pad = jnp.pad(part2, ((0, 0), (0, TROW - N * D)))

    gpad = _gather(idx2, part_pad)
    out = _add_noise(gpad, noise.reshape(B, N * D))
    return out.reshape(B, NG, PPG, D)
